# Initial kernel scaffold; baseline (speedup 1.0000x reference)
import jax
import jax.numpy as jnp
from jax.experimental import pallas as pl


def kernel(prev_embeddings, edges_ij, node_features_embeddings, edge_features_embeddings, W2):
    raise NotImplementedError("write your pallas kernel here")


# speedup vs baseline: 4.3321x; 4.3321x over previous
import functools

import jax
import jax.numpy as jnp
from jax import lax
from jax.experimental import pallas as pl
from jax.experimental.pallas import tpu as pltpu
from jax.experimental.pallas import tpu_sc as plsc

CH = 128


def _sc_scatter(prev_pad, srcs, dsts, n_nodes, d, nc, ns, chunks_per_tile):
  nw = nc * ns
  pt = chunks_per_tile * CH
  blk = 80
  nblocks = n_nodes // blk
  assert n_nodes % blk == 0 and blk % 16 == 0

  mesh = plsc.VectorSubcoreMesh(core_axis_name="c", subcore_axis_name="s")

  @functools.partial(
      pl.kernel,
      out_type=jax.ShapeDtypeStruct((nc * n_nodes, d), jnp.float32),
      mesh=mesh,
      scratch_types=[
          pltpu.VMEM((CH,), jnp.int32),
          pltpu.VMEM((CH,), jnp.int32),
          pltpu.VMEM((CH, d), jnp.float32),
          pltpu.VMEM((16, d), jnp.float32),
          pltpu.VMEM_SHARED((n_nodes, d), jnp.float32),
          pltpu.SemaphoreType.DMA,
      ],
  )
  def body(prev_hbm, srcs_hbm, dsts_hbm, out_hbm, sidx, didx, rows, zbuf, acc, sem):
    cid = lax.axis_index("c")
    sid = lax.axis_index("s")
    wid = sid * nc + cid
    my_nblk = (nblocks - 1 - sid) // ns + 1

    zvec = jnp.zeros((16,), jnp.float32)
    for r in range(16):
      for c in range(d // 16):
        zbuf[r, pl.ds(c * 16, 16)] = zvec

    def zero_body(j, carry):
      base = (sid + j * ns) * blk
      for k in range(blk // 16):
        pltpu.sync_copy(zbuf, acc.at[pl.ds(base + k * 16, 16)])
      return carry

    lax.fori_loop(0, my_nblk, zero_body, 0)
    plsc.subcore_barrier()

    def chunk_body(i, carry):
      base = wid * pt + i * CH
      pltpu.sync_copy(srcs_hbm.at[pl.ds(base, CH)], sidx)
      pltpu.sync_copy(dsts_hbm.at[pl.ds(base, CH)], didx)
      pltpu.async_copy(prev_hbm.at[sidx], rows, sem).wait()
      pltpu.sync_copy(rows, acc.at[didx], add=True)
      return carry

    lax.fori_loop(0, chunks_per_tile, chunk_body, 0)
    plsc.subcore_barrier()

    def wr_body(j, carry):
      base = (sid + j * ns) * blk
      pltpu.sync_copy(acc.at[pl.ds(base, blk)],
                      out_hbm.at[pl.ds(cid * n_nodes + base, blk)])
      return carry

    lax.fori_loop(0, my_nblk, wr_body, 0)

  return body(prev_pad, srcs, dsts)


def _tc_finish(p0, p1, nodef, edgef, w2, n_nodes, d):
  bn = 400
  grid = n_nodes // bn

  def body(p0_ref, p1_ref, nf_ref, ef_ref, w2_ref, out_ref):
    nbr = p0_ref[...] + p1_ref[...]
    x2 = lax.dot_general(
        nbr, w2_ref[...],
        dimension_numbers=(((1,), (1,)), ((), ())),
        preferred_element_type=jnp.float32,
    )
    x = nf_ref[...] + ef_ref[...] + x2
    out_ref[...] = jnp.where(x >= 0, x, 0.01 * x)

  row_spec = pl.BlockSpec((bn, d), lambda i: (i, 0))
  return pl.pallas_call(
      body,
      grid=(grid,),
      in_specs=[row_spec, row_spec, row_spec, row_spec,
                pl.BlockSpec((d, d), lambda i: (0, 0))],
      out_specs=row_spec,
      out_shape=jax.ShapeDtypeStruct((n_nodes, d), jnp.float32),
  )(p0, p1, nodef, edgef, w2)


def kernel(prev_embeddings, edges_ij, node_features_embeddings, edge_features_embeddings, W2):
  b, n, d = prev_embeddings.shape
  e = edges_ij.shape[0]

  info = plsc.get_sparse_core_info()
  nc, ns = info.num_cores, info.num_subcores
  nw = nc * ns

  chunks_per_tile = -(-2 * e // (nw * CH))
  pe = nw * chunks_per_tile * CH

  u = edges_ij[:, 0]
  v = edges_ij[:, 1]
  pad = pe - 2 * e
  srcs = jnp.concatenate([v, u, jnp.full((pad,), n, jnp.int32)])
  dsts = jnp.concatenate([u, v, jnp.zeros((pad,), jnp.int32)])
  prev_pad = jnp.concatenate(
      [prev_embeddings[0], jnp.zeros((8, d), jnp.float32)], axis=0)

  partials = _sc_scatter(prev_pad, srcs, dsts, n, d, nc, ns, chunks_per_tile)
  p0 = partials[:n]
  p1 = partials[n:]

  out = _tc_finish(p0, p1, node_features_embeddings[0],
                   edge_features_embeddings[0], W2, n, d)
  return out.reshape(b, n, d)
